# baseline (speedup 1.0000x reference)
import functools

import jax
import jax.numpy as jnp
from jax import lax
from jax.experimental import pallas as pl
from jax.experimental.pallas import tpu as pltpu
from jax.experimental.pallas import tpu_sc as plsc

N = 10000
E = 320000
D = 128
H = 16

NC = 2
NS = 16
L = 16
NW = NC * NS
EPW = E // NW
RPT = N // NS
CH = 80
NCH = EPW // CH
RING = 10
CPV = CH // L
RW = 640
RSTEP = 624

_mesh = plsc.VectorSubcoreMesh(
    core_axis_name="c", subcore_axis_name="s", num_cores=NC, num_subcores=NS
)
_sc_params = pltpu.CompilerParams(
    use_tc_tiling_on_sc=False, needs_layout_passes=False
)


def _worker_id():
    return lax.axis_index("s") * NC + lax.axis_index("c")


def _cross_tile_reduce(stage_sp, redbuf, outbuf, start):
    pltpu.sync_copy(stage_sp.at[:, pl.ds(start, RW)], redbuf)

    def red_body(i, carry):
        v = redbuf[0, pl.ds(i * L, L)]
        for j in range(1, NS):
            v = v + redbuf[j, pl.ds(i * L, L)]
        outbuf[pl.ds(i * L, L)] = v
        return carry

    lax.fori_loop(0, RW // L, red_body, 0, unroll=2)


@functools.partial(
    pl.kernel,
    out_type=jax.ShapeDtypeStruct((NC, 2, N), jnp.float32),
    mesh=_mesh,
    compiler_params=_sc_params,
    scratch_types=[
        pltpu.VMEM((NCH, CH), jnp.int32),
        pltpu.VMEM((NCH, CH), jnp.int32),
        pltpu.VMEM((N,), jnp.float32),
        pltpu.VMEM((N,), jnp.float32),
        pltpu.VMEM((NS, RW), jnp.float32),
        pltpu.VMEM((RW,), jnp.float32),
        pltpu.VMEM_SHARED((NS, N), jnp.float32),
        pltpu.VMEM_SHARED((NS, N), jnp.float32),
    ],
)
def _sc_degrees(
    esh_hbm, out_hbm, src_v, dst_v, hist_s, hist_d, redbuf, outbuf, hs_sp,
    hd_sp,
):
    c = lax.axis_index("c")
    s = lax.axis_index("s")
    w = s * NC + c
    pltpu.sync_copy(esh_hbm.at[0, pl.ds(w * NCH, NCH)], src_v)
    pltpu.sync_copy(esh_hbm.at[1, pl.ds(w * NCH, NCH)], dst_v)

    zeros = jnp.zeros((L,), jnp.float32)

    def zero_body(i, carry):
        hist_s[pl.ds(i * L, L)] = zeros
        hist_d[pl.ds(i * L, L)] = zeros
        return carry

    lax.fori_loop(0, N // L, zero_body, 0, unroll=8)

    ones = jnp.ones((L,), jnp.float32)

    def body(r, carry):
        for j in range(CPV):
            si = src_v[r, pl.ds(j * L, L)]
            plsc.addupdate_scatter(hist_s, [si], ones)
            di = dst_v[r, pl.ds(j * L, L)]
            plsc.addupdate_scatter(hist_d, [di], ones)
        return carry

    lax.fori_loop(0, NCH, body, 0, unroll=4)

    pltpu.sync_copy(hist_s, hs_sp.at[s])
    pltpu.sync_copy(hist_d, hd_sp.at[s])
    plsc.subcore_barrier()

    start = s * RSTEP
    _cross_tile_reduce(hs_sp, redbuf, outbuf, start)
    pltpu.sync_copy(outbuf, out_hbm.at[c, 0, pl.ds(start, RW)])
    _cross_tile_reduce(hd_sp, redbuf, outbuf, start)
    pltpu.sync_copy(outbuf, out_hbm.at[c, 1, pl.ds(start, RW)])


def _fast_rsqrt(x):
    i = plsc.bitcast(x, jnp.int32)
    i = 0x5F3759DF - lax.shift_right_arithmetic(i, 1)
    y = plsc.bitcast(i, jnp.float32)
    for _ in range(3):
        y = y * (1.5 - 0.5 * x * y * y)
    return y


@functools.partial(
    pl.kernel,
    out_type=(
        jax.ShapeDtypeStruct((NC, N, H), jnp.float32),
        jax.ShapeDtypeStruct((N, H), jnp.float32),
        jax.ShapeDtypeStruct((N,), jnp.float32),
        jax.ShapeDtypeStruct((N,), jnp.float32),
    ),
    mesh=_mesh,
    compiler_params=_sc_params,
    scratch_types=[
        pltpu.VMEM((NCH, CH), jnp.int32),
        pltpu.VMEM((NCH, CH), jnp.int32),
        pltpu.VMEM((RING, CH, H), jnp.float32),
        pltpu.VMEM((RPT, H), jnp.float32),
        pltpu.VMEM((RW, H), jnp.float32),
        pltpu.VMEM((RW,), jnp.float32),
        pltpu.VMEM((RW,), jnp.float32),
        pltpu.VMEM((RW,), jnp.float32),
        pltpu.VMEM_SHARED((N, H), jnp.float32),
        pltpu.VMEM_SHARED((N, H), jnp.float32),
        pltpu.SemaphoreType.DMA((RING,)),
        pltpu.SemaphoreType.DMA((RING,)),
    ],
)
def _sc_agg_rows(
    h0_hbm, degp_hbm, esh_hbm, out_hbm, g0out_hbm, dii_hbm,
    dio_hbm, idx_sv, idx_dv, rows, zbuf, h0_v, dii_v, dio_v, dtmp_v, acc_sp,
    g0_sp, gsem, ssem,
):
    c = lax.axis_index("c")
    s = lax.axis_index("s")
    w = s * NC + c
    start = s * RSTEP

    pltpu.sync_copy(esh_hbm.at[0, pl.ds(w * NCH, NCH)], idx_sv)
    pltpu.sync_copy(esh_hbm.at[1, pl.ds(w * NCH, NCH)], idx_dv)

    pltpu.sync_copy(degp_hbm.at[0, 0, pl.ds(start, RW)], dio_v)
    pltpu.sync_copy(degp_hbm.at[1, 0, pl.ds(start, RW)], dtmp_v)

    def dio_body(t, carry):
        d = dio_v[pl.ds(t * L, L)] + dtmp_v[pl.ds(t * L, L)] + 1.0
        dio_v[pl.ds(t * L, L)] = _fast_rsqrt(d)
        return carry

    lax.fori_loop(0, RW // L, dio_body, 0, unroll=2)

    pltpu.sync_copy(degp_hbm.at[0, 1, pl.ds(start, RW)], dii_v)
    pltpu.sync_copy(degp_hbm.at[1, 1, pl.ds(start, RW)], dtmp_v)

    def dii_body(t, carry):
        d = dii_v[pl.ds(t * L, L)] + dtmp_v[pl.ds(t * L, L)] + 1.0
        dii_v[pl.ds(t * L, L)] = _fast_rsqrt(d)
        return carry

    lax.fori_loop(0, RW // L, dii_body, 0, unroll=2)

    pltpu.sync_copy(dio_v, dio_hbm.at[pl.ds(start, RW)])
    pltpu.sync_copy(dii_v, dii_hbm.at[pl.ds(start, RW)])

    pltpu.sync_copy(h0_hbm.at[pl.ds(start, RW)], h0_v)

    def scale_body(t, carry):
        dv = dio_v[pl.ds(t * L, L)]
        for ln in range(L):
            i = t * L + ln
            h0_v[i, :] = h0_v[i, :] * dv[ln]
        return carry

    lax.fori_loop(0, RW // L, scale_body, 0)

    pltpu.sync_copy(h0_v, g0_sp.at[pl.ds(start, RW)])
    pltpu.sync_copy(h0_v, g0out_hbm.at[pl.ds(start, RW)])

    zeros = jnp.zeros((L,), jnp.float32)

    def zero_body(i, carry):
        zbuf[i, :] = zeros
        return carry

    lax.fori_loop(0, RPT, zero_body, 0, unroll=8)
    pltpu.sync_copy(zbuf, acc_sp.at[pl.ds(s * RPT, RPT)])
    plsc.subcore_barrier()

    def _wait_gather(k, j):
        pltpu.make_async_copy(
            g0_sp.at[idx_sv.at[k]], rows.at[j], gsem.at[j]
        ).wait()

    def _scatter(k, j):
        pltpu.async_copy(
            rows.at[j], acc_sp.at[idx_dv.at[k]], ssem.at[j], add=True
        )

    def _wait_scatter(k, j):
        pltpu.make_async_copy(
            rows.at[j], acc_sp.at[idx_dv.at[k]], ssem.at[j]
        ).wait()

    for j in range(RING):
        pltpu.async_copy(g0_sp.at[idx_sv.at[j]], rows.at[j], gsem.at[j])

    NIT = (NCH - RING) // RING
    TAIL = NCH - RING * (NIT + 1)

    def ring_body(i, carry):
        for j in range(RING):
            k = i * RING + j
            _wait_gather(k, j)
            _scatter(k, j)
            _wait_scatter(k, j)
            pltpu.async_copy(
                g0_sp.at[idx_sv.at[k + RING]], rows.at[j], gsem.at[j]
            )
        return carry

    lax.fori_loop(0, NIT, ring_body, 0)

    for j in range(RING):
        k = NIT * RING + j
        _wait_gather(k, j)
        _scatter(k, j)
        _wait_scatter(k, j)
        if j < TAIL:
            kk = (NIT + 1) * RING + j
            pltpu.async_copy(
                g0_sp.at[idx_sv.at[kk]], rows.at[j], gsem.at[j]
            )
    for j in range(TAIL):
        k = (NIT + 1) * RING + j
        _wait_gather(k, j)
        _scatter(k, j)
        _wait_scatter(k, j)
    plsc.subcore_barrier()

    pltpu.sync_copy(
        acc_sp.at[pl.ds(s * RPT, RPT)], out_hbm.at[c, pl.ds(s * RPT, RPT)]
    )


@functools.partial(
    pl.kernel,
    out_type=(
        jax.ShapeDtypeStruct((NC, N), jnp.float32),
        jax.ShapeDtypeStruct((N,), jnp.float32),
    ),
    mesh=_mesh,
    compiler_params=_sc_params,
    scratch_types=[
        pltpu.VMEM((N,), jnp.float32),
        pltpu.VMEM((NCH, CH), jnp.int32),
        pltpu.VMEM((NCH, CH), jnp.int32),
        pltpu.VMEM((N,), jnp.float32),
        pltpu.VMEM((NS, RW), jnp.float32),
        pltpu.VMEM((RW,), jnp.float32),
        pltpu.VMEM_SHARED((NS, N), jnp.float32),
        pltpu.VMEM((RW, H), jnp.float32),
        pltpu.VMEM((RW, H), jnp.float32),
        pltpu.VMEM((RW, H), jnp.float32),
        pltpu.VMEM((RW,), jnp.float32),
        pltpu.VMEM((RW,), jnp.float32),
        pltpu.VMEM((RW,), jnp.float32),
        pltpu.VMEM((H,), jnp.float32),
        pltpu.VMEM((H,), jnp.float32),
        pltpu.VMEM_SHARED((N,), jnp.float32),
    ],
)
def _sc_agg_scalar(
    accp_hbm, g0_hbm, dii_hbm, dio_hbm, b0_hbm, w1_hbm, esh_hbm,
    out_hbm, g1out_hbm, g1_v, idx_s2, idx_d2, acc_v, redbuf, outbuf, acc_sp,
    a0_v, a1_v, gg_v, di_v, do_v, g1s_v, b0_v, w1_v, g1_sp,
):
    c = lax.axis_index("c")
    s = lax.axis_index("s")
    w = s * NC + c
    start = s * RSTEP

    pltpu.sync_copy(accp_hbm.at[0, pl.ds(start, RW)], a0_v)
    pltpu.sync_copy(accp_hbm.at[1, pl.ds(start, RW)], a1_v)
    pltpu.sync_copy(g0_hbm.at[pl.ds(start, RW)], gg_v)
    pltpu.sync_copy(dii_hbm.at[pl.ds(start, RW)], di_v)
    pltpu.sync_copy(dio_hbm.at[pl.ds(start, RW)], do_v)
    pltpu.sync_copy(b0_hbm, b0_v)
    pltpu.sync_copy(w1_hbm, w1_v)
    pltpu.sync_copy(esh_hbm.at[0, pl.ds(w * NCH, NCH)], idx_s2)
    pltpu.sync_copy(esh_hbm.at[1, pl.ds(w * NCH, NCH)], idx_d2)

    b0r = b0_v[...]
    w1r = w1_v[...]
    lane = lax.iota(jnp.int32, L)

    def g1_body(t, carry):
        base_i = t * L
        div16 = di_v[pl.ds(base_i, L)]
        dov16 = do_v[pl.ds(base_i, L)]
        zvec = jnp.zeros((L,), jnp.float32)
        for ln in range(L):
            i = base_i + ln
            row = a0_v[i, :] + a1_v[i, :] + gg_v[i, :]
            pre = row * div16[ln] + b0r
            h1 = jnp.maximum(pre, 0.0)
            zs = jnp.sum(h1 * w1r)
            zvec = jnp.where(lane == ln, zs, zvec)
        g1s_v[pl.ds(base_i, L)] = zvec * dov16
        return carry

    lax.fori_loop(0, RW // L, g1_body, 0)

    pltpu.sync_copy(g1s_v, g1_sp.at[pl.ds(start, RW)])
    pltpu.sync_copy(g1s_v, g1out_hbm.at[pl.ds(start, RW)])
    plsc.subcore_barrier()
    pltpu.sync_copy(g1_sp, g1_v)

    zeros = jnp.zeros((L,), jnp.float32)

    def zero_body(i, carry):
        acc_v[pl.ds(i * L, L)] = zeros
        return carry

    lax.fori_loop(0, N // L, zero_body, 0, unroll=8)

    def body(r, carry):
        for j in range(CPV):
            iv = idx_s2[r, pl.ds(j * L, L)]
            vals = plsc.load_gather(g1_v, [iv])
            jv = idx_d2[r, pl.ds(j * L, L)]
            plsc.addupdate_scatter(acc_v, [jv], vals)
        return carry

    lax.fori_loop(0, NCH, body, 0, unroll=4)

    pltpu.sync_copy(acc_v, acc_sp.at[s])
    plsc.subcore_barrier()

    _cross_tile_reduce(acc_sp, redbuf, outbuf, start)
    pltpu.sync_copy(outbuf, out_hbm.at[c, pl.ds(start, RW)])


def _tc0_body(x_ref, w0_ref, h0_ref):
    h0_ref[...] = jnp.dot(
        x_ref[...], w0_ref[...], preferred_element_type=jnp.float32
    )


def _tc0(x, w0):
    return pl.pallas_call(
        _tc0_body,
        out_shape=jax.ShapeDtypeStruct((N, H), jnp.float32),
    )(x, w0)


def _tc3_body(accp_ref, g1_ref, dii_ref, b1_ref, out_ref):
    acc = accp_ref[0] + accp_ref[1] + g1_ref[...]
    pre = acc * dii_ref[...] + b1_ref[0]
    out_ref[...] = jax.nn.sigmoid(pre)[:, None]


def _tc3(accp, g1, dinv_in, b1):
    return pl.pallas_call(
        _tc3_body,
        out_shape=jax.ShapeDtypeStruct((N, 1), jnp.float32),
    )(accp, g1, dinv_in, b1)


def kernel(x, edge_index, W0, b0, W1, b1):
    esh = edge_index.reshape(2, NW * NCH, CH)
    degp = _sc_degrees(esh)
    h0 = _tc0(x, W0)
    accp0, g0, dinv_in, dinv_out = _sc_agg_rows(h0, degp, esh)
    accp1, g1 = _sc_agg_scalar(
        accp0, g0, dinv_in, dinv_out, b0, W1.reshape(H), esh
    )
    return _tc3(accp1, g1, dinv_in, b1)


# speedup vs baseline: 1.1010x; 1.1010x over previous
import functools

import jax
import jax.numpy as jnp
from jax import lax
from jax.experimental import pallas as pl
from jax.experimental.pallas import tpu as pltpu
from jax.experimental.pallas import tpu_sc as plsc

N = 10000
E = 320000
D = 128
H = 16

NC = 2
NS = 16
L = 16
NW = NC * NS
EPW = E // NW
RPT = N // NS
CH = 80
NCH = EPW // CH
RING = 5
CPV = CH // L
RW = 640
RSTEP = 624

_mesh = plsc.VectorSubcoreMesh(
    core_axis_name="c", subcore_axis_name="s", num_cores=NC, num_subcores=NS
)
_sc_params = pltpu.CompilerParams(
    use_tc_tiling_on_sc=False, needs_layout_passes=False
)


def _worker_id():
    return lax.axis_index("s") * NC + lax.axis_index("c")


def _cross_tile_reduce(stage_sp, redbuf, outbuf, start):
    pltpu.sync_copy(stage_sp.at[:, pl.ds(start, RW)], redbuf)

    def red_body(i, carry):
        v = redbuf[0, pl.ds(i * L, L)]
        for j in range(1, NS):
            v = v + redbuf[j, pl.ds(i * L, L)]
        outbuf[pl.ds(i * L, L)] = v
        return carry

    lax.fori_loop(0, RW // L, red_body, 0, unroll=2)


@functools.partial(
    pl.kernel,
    out_type=jax.ShapeDtypeStruct((NC, 2, N), jnp.float32),
    mesh=_mesh,
    compiler_params=_sc_params,
    scratch_types=[
        pltpu.VMEM((NCH, CH), jnp.int32),
        pltpu.VMEM((NCH, CH), jnp.int32),
        pltpu.VMEM((N,), jnp.float32),
        pltpu.VMEM((N,), jnp.float32),
        pltpu.VMEM((NS, RW), jnp.float32),
        pltpu.VMEM((RW,), jnp.float32),
        pltpu.VMEM_SHARED((NS, N), jnp.float32),
        pltpu.VMEM_SHARED((NS, N), jnp.float32),
    ],
)
def _sc_degrees(
    esh_hbm, out_hbm, src_v, dst_v, hist_s, hist_d, redbuf, outbuf, hs_sp,
    hd_sp,
):
    c = lax.axis_index("c")
    s = lax.axis_index("s")
    w = s * NC + c
    pltpu.sync_copy(esh_hbm.at[0, pl.ds(w * NCH, NCH)], src_v)
    pltpu.sync_copy(esh_hbm.at[1, pl.ds(w * NCH, NCH)], dst_v)

    zeros = jnp.zeros((L,), jnp.float32)

    def zero_body(i, carry):
        hist_s[pl.ds(i * L, L)] = zeros
        hist_d[pl.ds(i * L, L)] = zeros
        return carry

    lax.fori_loop(0, N // L, zero_body, 0, unroll=8)

    ones = jnp.ones((L,), jnp.float32)

    def body(r, carry):
        for j in range(CPV):
            si = src_v[r, pl.ds(j * L, L)]
            plsc.addupdate_scatter(hist_s, [si], ones)
            di = dst_v[r, pl.ds(j * L, L)]
            plsc.addupdate_scatter(hist_d, [di], ones)
        return carry

    lax.fori_loop(0, NCH, body, 0, unroll=2)

    pltpu.sync_copy(hist_s, hs_sp.at[s])
    pltpu.sync_copy(hist_d, hd_sp.at[s])
    plsc.subcore_barrier()

    start = s * RSTEP
    _cross_tile_reduce(hs_sp, redbuf, outbuf, start)
    pltpu.sync_copy(outbuf, out_hbm.at[c, 0, pl.ds(start, RW)])
    _cross_tile_reduce(hd_sp, redbuf, outbuf, start)
    pltpu.sync_copy(outbuf, out_hbm.at[c, 1, pl.ds(start, RW)])


def _fast_rsqrt(x):
    i = plsc.bitcast(x, jnp.int32)
    i = 0x5F3759DF - lax.shift_right_arithmetic(i, 1)
    y = plsc.bitcast(i, jnp.float32)
    for _ in range(3):
        y = y * (1.5 - 0.5 * x * y * y)
    return y


@functools.partial(
    pl.kernel,
    out_type=(
        jax.ShapeDtypeStruct((NC, N, H), jnp.float32),
        jax.ShapeDtypeStruct((N, H), jnp.float32),
        jax.ShapeDtypeStruct((N,), jnp.float32),
        jax.ShapeDtypeStruct((N,), jnp.float32),
    ),
    mesh=_mesh,
    compiler_params=_sc_params,
    scratch_types=[
        pltpu.VMEM((NCH, CH), jnp.int32),
        pltpu.VMEM((NCH, CH), jnp.int32),
        pltpu.VMEM((RING, CH, H), jnp.float32),
        pltpu.VMEM((RPT, H), jnp.float32),
        pltpu.VMEM((RW, H), jnp.float32),
        pltpu.VMEM((RW,), jnp.float32),
        pltpu.VMEM((RW,), jnp.float32),
        pltpu.VMEM((RW,), jnp.float32),
        pltpu.VMEM((RW,), jnp.float32),
        pltpu.VMEM_SHARED((N, H), jnp.float32),
        pltpu.VMEM_SHARED((N, H), jnp.float32),
        pltpu.SemaphoreType.DMA((RING,)),
        pltpu.SemaphoreType.DMA((RING,)),
    ],
)
def _sc_agg_rows(
    h0_hbm, degp_hbm, esh_hbm, out_hbm, g0out_hbm, dii_hbm,
    dio_hbm, idx_sv, idx_dv, rows, zbuf, h0_v, dii_v, dio_v, dtmp_v, dtmp2_v,
    acc_sp, g0_sp, gsem, ssem,
):
    c = lax.axis_index("c")
    s = lax.axis_index("s")
    w = s * NC + c
    start = s * RSTEP

    ld = [
        pltpu.async_copy(
            esh_hbm.at[0, pl.ds(w * NCH, NCH)], idx_sv, gsem.at[0]
        ),
        pltpu.async_copy(
            esh_hbm.at[1, pl.ds(w * NCH, NCH)], idx_dv, gsem.at[1]
        ),
        pltpu.async_copy(h0_hbm.at[pl.ds(start, RW)], h0_v, gsem.at[2]),
    ]
    dld = [
        pltpu.async_copy(
            degp_hbm.at[0, 0, pl.ds(start, RW)], dio_v, ssem.at[0]
        ),
        pltpu.async_copy(
            degp_hbm.at[1, 0, pl.ds(start, RW)], dtmp_v, ssem.at[1]
        ),
    ]

    zeros = jnp.zeros((L,), jnp.float32)

    def zero_body(i, carry):
        zbuf[i, :] = zeros
        return carry

    lax.fori_loop(0, RPT, zero_body, 0, unroll=8)
    for d in dld:
        d.wait()

    def dio_body(t, carry):
        d = dio_v[pl.ds(t * L, L)] + dtmp_v[pl.ds(t * L, L)] + 1.0
        dio_v[pl.ds(t * L, L)] = _fast_rsqrt(d)
        return carry

    dld = [
        pltpu.async_copy(
            degp_hbm.at[0, 1, pl.ds(start, RW)], dii_v, ssem.at[0]
        ),
        pltpu.async_copy(
            degp_hbm.at[1, 1, pl.ds(start, RW)], dtmp2_v, ssem.at[1]
        ),
    ]
    lax.fori_loop(0, RW // L, dio_body, 0, unroll=2)
    for d in dld:
        d.wait()

    def dii_body(t, carry):
        d = dii_v[pl.ds(t * L, L)] + dtmp2_v[pl.ds(t * L, L)] + 1.0
        dii_v[pl.ds(t * L, L)] = _fast_rsqrt(d)
        return carry

    lax.fori_loop(0, RW // L, dii_body, 0, unroll=2)

    pltpu.sync_copy(dio_v, dio_hbm.at[pl.ds(start, RW)])
    pltpu.sync_copy(dii_v, dii_hbm.at[pl.ds(start, RW)])

    ld[2].wait()

    def scale_body(t, carry):
        dv = dio_v[pl.ds(t * L, L)]
        for ln in range(L):
            i = t * L + ln
            h0_v[i, :] = h0_v[i, :] * dv[ln]
        return carry

    lax.fori_loop(0, RW // L, scale_body, 0)

    pltpu.sync_copy(h0_v, g0_sp.at[pl.ds(start, RW)])
    pltpu.sync_copy(h0_v, g0out_hbm.at[pl.ds(start, RW)])
    pltpu.sync_copy(zbuf, acc_sp.at[pl.ds(s * RPT, RPT)])
    ld[0].wait()
    ld[1].wait()
    plsc.subcore_barrier()

    def _wait_gather(k, j):
        pltpu.make_async_copy(
            g0_sp.at[idx_sv.at[k]], rows.at[j], gsem.at[j]
        ).wait()

    def _scatter(k, j):
        pltpu.async_copy(
            rows.at[j], acc_sp.at[idx_dv.at[k]], ssem.at[j], add=True
        )

    def _wait_scatter(k, j):
        pltpu.make_async_copy(
            rows.at[j], acc_sp.at[idx_dv.at[k]], ssem.at[j]
        ).wait()

    for j in range(RING):
        pltpu.async_copy(g0_sp.at[idx_sv.at[j]], rows.at[j], gsem.at[j])

    NIT = (NCH - RING) // RING
    TAIL = NCH - RING * (NIT + 1)

    def ring_body(i, carry):
        for j in range(RING):
            k = i * RING + j
            _wait_gather(k, j)
            _scatter(k, j)
            _wait_scatter(k, j)
            pltpu.async_copy(
                g0_sp.at[idx_sv.at[k + RING]], rows.at[j], gsem.at[j]
            )
        return carry

    lax.fori_loop(0, NIT, ring_body, 0)

    for j in range(RING):
        k = NIT * RING + j
        _wait_gather(k, j)
        _scatter(k, j)
        _wait_scatter(k, j)
        if j < TAIL:
            kk = (NIT + 1) * RING + j
            pltpu.async_copy(
                g0_sp.at[idx_sv.at[kk]], rows.at[j], gsem.at[j]
            )
    for j in range(TAIL):
        k = (NIT + 1) * RING + j
        _wait_gather(k, j)
        _scatter(k, j)
        _wait_scatter(k, j)
    plsc.subcore_barrier()

    pltpu.sync_copy(
        acc_sp.at[pl.ds(s * RPT, RPT)], out_hbm.at[c, pl.ds(s * RPT, RPT)]
    )


@functools.partial(
    pl.kernel,
    out_type=(
        jax.ShapeDtypeStruct((NC, N), jnp.float32),
        jax.ShapeDtypeStruct((N,), jnp.float32),
    ),
    mesh=_mesh,
    compiler_params=_sc_params,
    scratch_types=[
        pltpu.VMEM((N,), jnp.float32),
        pltpu.VMEM((NCH, CH), jnp.int32),
        pltpu.VMEM((NCH, CH), jnp.int32),
        pltpu.VMEM((N,), jnp.float32),
        pltpu.VMEM((NS, RW), jnp.float32),
        pltpu.VMEM((RW,), jnp.float32),
        pltpu.VMEM_SHARED((NS, N), jnp.float32),
        pltpu.VMEM((RW, H), jnp.float32),
        pltpu.VMEM((RW, H), jnp.float32),
        pltpu.VMEM((RW, H), jnp.float32),
        pltpu.VMEM((RW,), jnp.float32),
        pltpu.VMEM((RW,), jnp.float32),
        pltpu.VMEM((RW,), jnp.float32),
        pltpu.VMEM((H,), jnp.float32),
        pltpu.VMEM((H,), jnp.float32),
        pltpu.VMEM_SHARED((N,), jnp.float32),
        pltpu.SemaphoreType.DMA((9,)),
    ],
)
def _sc_agg_scalar(
    accp_hbm, g0_hbm, dii_hbm, dio_hbm, b0_hbm, w1_hbm, esh_hbm,
    out_hbm, g1out_hbm, g1_v, idx_s2, idx_d2, acc_v, redbuf, outbuf, acc_sp,
    a0_v, a1_v, gg_v, di_v, do_v, g1s_v, b0_v, w1_v, g1_sp, lsem,
):
    c = lax.axis_index("c")
    s = lax.axis_index("s")
    w = s * NC + c
    start = s * RSTEP

    ld = [
        pltpu.async_copy(accp_hbm.at[0, pl.ds(start, RW)], a0_v, lsem.at[0]),
        pltpu.async_copy(accp_hbm.at[1, pl.ds(start, RW)], a1_v, lsem.at[1]),
        pltpu.async_copy(g0_hbm.at[pl.ds(start, RW)], gg_v, lsem.at[2]),
        pltpu.async_copy(dii_hbm.at[pl.ds(start, RW)], di_v, lsem.at[3]),
        pltpu.async_copy(dio_hbm.at[pl.ds(start, RW)], do_v, lsem.at[4]),
        pltpu.async_copy(b0_hbm, b0_v, lsem.at[5]),
        pltpu.async_copy(w1_hbm, w1_v, lsem.at[6]),
        pltpu.async_copy(
            esh_hbm.at[0, pl.ds(w * NCH, NCH)], idx_s2, lsem.at[7]
        ),
        pltpu.async_copy(
            esh_hbm.at[1, pl.ds(w * NCH, NCH)], idx_d2, lsem.at[8]
        ),
    ]

    zeros = jnp.zeros((L,), jnp.float32)

    def zero_acc(i, carry):
        acc_v[pl.ds(i * L, L)] = zeros
        return carry

    lax.fori_loop(0, N // L, zero_acc, 0, unroll=8)
    for d in ld[:7]:
        d.wait()

    b0r = b0_v[...]
    w1r = w1_v[...]
    lane = lax.iota(jnp.int32, L)

    def g1_body(t, carry):
        base_i = t * L
        div16 = di_v[pl.ds(base_i, L)]
        dov16 = do_v[pl.ds(base_i, L)]
        zvec = jnp.zeros((L,), jnp.float32)
        for ln in range(L):
            i = base_i + ln
            row = a0_v[i, :] + a1_v[i, :] + gg_v[i, :]
            pre = row * div16[ln] + b0r
            h1 = jnp.maximum(pre, 0.0)
            zs = jnp.sum(h1 * w1r)
            zvec = jnp.where(lane == ln, zs, zvec)
        g1s_v[pl.ds(base_i, L)] = zvec * dov16
        return carry

    lax.fori_loop(0, RW // L, g1_body, 0)

    pltpu.sync_copy(g1s_v, g1_sp.at[pl.ds(start, RW)])
    pltpu.sync_copy(g1s_v, g1out_hbm.at[pl.ds(start, RW)])
    ld[7].wait()
    ld[8].wait()
    plsc.subcore_barrier()
    pltpu.sync_copy(g1_sp, g1_v)

    def body(r, carry):
        for j in range(CPV):
            iv = idx_s2[r, pl.ds(j * L, L)]
            vals = plsc.load_gather(g1_v, [iv])
            jv = idx_d2[r, pl.ds(j * L, L)]
            plsc.addupdate_scatter(acc_v, [jv], vals)
        return carry

    lax.fori_loop(0, NCH, body, 0, unroll=2)

    pltpu.sync_copy(acc_v, acc_sp.at[s])
    plsc.subcore_barrier()

    _cross_tile_reduce(acc_sp, redbuf, outbuf, start)
    pltpu.sync_copy(outbuf, out_hbm.at[c, pl.ds(start, RW)])


def _tc0_body(x_ref, w0_ref, h0_ref):
    h0_ref[...] = jnp.dot(
        x_ref[...], w0_ref[...], preferred_element_type=jnp.float32
    )


def _tc0(x, w0):
    return pl.pallas_call(
        _tc0_body,
        out_shape=jax.ShapeDtypeStruct((N, H), jnp.float32),
    )(x, w0)


def _tc3_body(accp_ref, g1_ref, dii_ref, b1_ref, out_ref):
    acc = accp_ref[0] + accp_ref[1] + g1_ref[...]
    pre = acc * dii_ref[...] + b1_ref[0]
    out_ref[...] = jax.nn.sigmoid(pre)[:, None]


def _tc3(accp, g1, dinv_in, b1):
    return pl.pallas_call(
        _tc3_body,
        out_shape=jax.ShapeDtypeStruct((N, 1), jnp.float32),
    )(accp, g1, dinv_in, b1)


def kernel(x, edge_index, W0, b0, W1, b1):
    esh = edge_index.reshape(2, NW * NCH, CH)
    degp = _sc_degrees(esh)
    h0 = _tc0(x, W0)
    accp0, g0, dinv_in, dinv_out = _sc_agg_rows(h0, degp, esh)
    accp1, g1 = _sc_agg_scalar(
        accp0, g0, dinv_in, dinv_out, b0, W1.reshape(H), esh
    )
    return _tc3(accp1, g1, dinv_in, b1)
